# baseline (speedup 1.0000x reference)
import functools

import jax
import jax.numpy as jnp
from jax import lax
from jax.experimental import pallas as pl
from jax.experimental.pallas import tpu as pltpu
from jax.experimental.pallas import tpu_sc as plsc

_BATCH = 16384
_N_FIELDS = 26
_DIM = 64
_NUM_ROWS = _BATCH * _N_FIELDS

_NC = 2
_NS = 16
_NW = _NC * _NS

_ROWS_PER_W = _NUM_ROWS // _NW
assert _ROWS_PER_W * _NW == _NUM_ROWS

_CHUNK = 512
_N_CHUNKS = _ROWS_PER_W // _CHUNK
assert _N_CHUNKS * _CHUNK == _ROWS_PER_W


def _gather_body(idx_hbm, table_hbm, out_hbm, idx_v, rows_v, sem):
    wid = lax.axis_index("s") * _NC + lax.axis_index("c")
    base = wid * _ROWS_PER_W

    def body(i, carry):
        off = base + i * _CHUNK
        pltpu.sync_copy(idx_hbm.at[pl.ds(off, _CHUNK)], idx_v)
        pltpu.async_copy(table_hbm.at[idx_v], rows_v, sem).wait()
        pltpu.sync_copy(rows_v, out_hbm.at[pl.ds(off, _CHUNK)])
        return carry

    lax.fori_loop(0, _N_CHUNKS, body, 0)


@functools.partial(
    pl.kernel,
    mesh=plsc.VectorSubcoreMesh(core_axis_name="c", subcore_axis_name="s"),
    out_type=jax.ShapeDtypeStruct((_NUM_ROWS, _DIM), jnp.float32),
    scratch_types=[
        pltpu.VMEM((_CHUNK,), jnp.int32),
        pltpu.VMEM((_CHUNK, _DIM), jnp.float32),
        pltpu.SemaphoreType.DMA,
    ],
    compiler_params=pltpu.CompilerParams(use_tc_tiling_on_sc=False),
)
def _gather_call(idx_hbm, table_hbm, out_hbm, idx_v, rows_v, sem):
    _gather_body(idx_hbm, table_hbm, out_hbm, idx_v, rows_v, sem)


@jax.jit
def kernel(X, embeddings):
    idx = X.reshape(-1).astype(jnp.int32)
    out = _gather_call(idx, embeddings)
    return out.reshape(_BATCH, _N_FIELDS, _DIM)


# speedup vs baseline: 1.0235x; 1.0235x over previous
import functools

import jax
import jax.numpy as jnp
from jax import lax
from jax.experimental import pallas as pl
from jax.experimental.pallas import tpu as pltpu
from jax.experimental.pallas import tpu_sc as plsc

_BATCH = 16384
_N_FIELDS = 26
_DIM = 64
_NUM_ROWS = _BATCH * _N_FIELDS

_NC = 2
_NS = 16
_NW = _NC * _NS

_ROWS_PER_W = _NUM_ROWS // _NW
assert _ROWS_PER_W * _NW == _NUM_ROWS

_CHUNK = 416
_NBUF = 4
_N_CHUNKS = _ROWS_PER_W // _CHUNK
assert _N_CHUNKS * _CHUNK == _ROWS_PER_W
_N_GROUPS = _N_CHUNKS // _NBUF
assert _N_GROUPS * _NBUF == _N_CHUNKS


def _gather_body(idx_hbm, table_hbm, out_hbm, idx_v, rows_v, gsems, wsems):
    wid = lax.axis_index("s") * _NC + lax.axis_index("c")
    base = wid * _ROWS_PER_W

    def start_gather(b, chunk_i):
        off = base + chunk_i * _CHUNK
        pltpu.sync_copy(idx_hbm.at[pl.ds(off, _CHUNK)], idx_v.at[b])
        pltpu.async_copy(table_hbm.at[idx_v.at[b]], rows_v.at[b], gsems.at[b])

    def wait_gather(b):
        pltpu.make_async_copy(
            table_hbm.at[idx_v.at[b]], rows_v.at[b], gsems.at[b]
        ).wait()

    def start_writeback(b, chunk_i):
        off = base + chunk_i * _CHUNK
        pltpu.async_copy(rows_v.at[b], out_hbm.at[pl.ds(off, _CHUNK)], wsems.at[b])

    def wait_writeback(b):
        pltpu.make_async_copy(
            rows_v.at[b], out_hbm.at[pl.ds(base, _CHUNK)], wsems.at[b]
        ).wait()

    for b in range(_NBUF):
        start_gather(b, b)

    def group(j, carry):
        for b in range(_NBUF):
            i = j * _NBUF + b
            wait_gather(b)
            start_writeback(b, i)

            @pl.when(j < _N_GROUPS - 1)
            def _():
                wait_writeback(b)
                start_gather(b, i + _NBUF)

        return carry

    lax.fori_loop(0, _N_GROUPS, group, 0)

    for b in range(_NBUF):
        wait_writeback(b)


@functools.partial(
    pl.kernel,
    mesh=plsc.VectorSubcoreMesh(core_axis_name="c", subcore_axis_name="s"),
    out_type=jax.ShapeDtypeStruct((_NUM_ROWS, _DIM), jnp.float32),
    scratch_types=[
        pltpu.VMEM((_NBUF, _CHUNK), jnp.int32),
        pltpu.VMEM((_NBUF, _CHUNK, _DIM), jnp.float32),
        pltpu.SemaphoreType.DMA((_NBUF,)),
        pltpu.SemaphoreType.DMA((_NBUF,)),
    ],
    compiler_params=pltpu.CompilerParams(use_tc_tiling_on_sc=False),
)
def _gather_call(idx_hbm, table_hbm, out_hbm, idx_v, rows_v, gsems, wsems):
    _gather_body(idx_hbm, table_hbm, out_hbm, idx_v, rows_v, gsems, wsems)


@jax.jit
def kernel(X, embeddings):
    idx = X.reshape(-1).astype(jnp.int32)
    out = _gather_call(idx, embeddings)
    return out.reshape(_BATCH, _N_FIELDS, _DIM)
